# baseline (speedup 1.0000x reference)
import jax
import jax.numpy as jnp
from jax import lax
from jax.experimental import pallas as pl
from jax.experimental.pallas import tpu as pltpu
from jax.experimental.pallas import tpu_sc as plsc

_NC, _NS = 2, 16
_CHUNK = 288
_W = 8
_N = 50000
_E = 1600000
_NG = 128
_NP = 50048
_NCH = -(-(_E // _CHUNK) // 256) * 256
_PER = _NCH // (_NC * _NS)
_RPT = _NP // _NS
_K = 8
_RB = 544
_G = _NP // _RB


def _make_agg(n_chunks_feat):
    mesh = plsc.VectorSubcoreMesh(core_axis_name="c", subcore_axis_name="s")
    nf = n_chunks_feat

    def body(*refs):
        tabs = refs[:nf]
        src_h, dst_h, zeros_h = refs[nf:nf + 3]
        outs = refs[nf + 3:2 * nf + 3]
        src_v, dst_v, rows0, rows1, acc, sem0, sem1 = refs[2 * nf + 3:]

        cid = lax.axis_index("c")
        sid = lax.axis_index("s")
        wid = sid * _NC + cid

        pltpu.sync_copy(src_h.at[pl.ds(wid * _PER, _PER)], src_v)
        pltpu.sync_copy(dst_h.at[pl.ds(wid * _PER, _PER)], dst_v)

        for ci in range(nf):
            pltpu.sync_copy(zeros_h, acc.at[pl.ds(sid * _RPT, _RPT)])
            plsc.subcore_barrier()

            pltpu.async_copy(tabs[ci].at[src_v.at[0]], rows0, sem0)

            @pl.loop(0, _PER, step=2)
            def _edges(j):
                pltpu.make_async_copy(
                    tabs[ci].at[src_v.at[j]], rows0, sem0).wait()
                pltpu.async_copy(tabs[ci].at[src_v.at[j + 1]], rows1, sem1)
                pltpu.sync_copy(rows0, acc.at[dst_v.at[j]], add=True)
                pltpu.make_async_copy(
                    tabs[ci].at[src_v.at[j + 1]], rows1, sem1).wait()

                @pl.when(j + 2 < _PER)
                def _next():
                    pltpu.async_copy(
                        tabs[ci].at[src_v.at[j + 2]], rows0, sem0)
                pltpu.sync_copy(rows1, acc.at[dst_v.at[j + 1]], add=True)
            plsc.subcore_barrier()

            pltpu.sync_copy(
                acc.at[pl.ds(sid * _RPT, _RPT)],
                outs[ci].at[cid, pl.ds(sid * _RPT, _RPT)])
            plsc.subcore_barrier()

    return pl.kernel(
        body,
        out_type=[jax.ShapeDtypeStruct((_NC, _NP, _W), jnp.float32)
                  for _ in range(nf)],
        mesh=mesh,
        compiler_params=pltpu.CompilerParams(use_tc_tiling_on_sc=False),
        scratch_types=[
            pltpu.VMEM((_PER, _CHUNK), jnp.int32),
            pltpu.VMEM((_PER, _CHUNK), jnp.int32),
            pltpu.VMEM((_CHUNK, _W), jnp.float32),
            pltpu.VMEM((_CHUNK, _W), jnp.float32),
            pltpu.VMEM_SHARED((_NP, _W), jnp.float32),
            pltpu.SemaphoreType.DMA,
            pltpu.SemaphoreType.DMA,
        ],
    )


def _make_deg():
    mesh = plsc.VectorSubcoreMesh(core_axis_name="c", subcore_axis_name="s")

    def body(dst_h, ones_h, zeros_h, out, dst_v, ones_v, acc, sem):
        del sem
        cid = lax.axis_index("c")
        sid = lax.axis_index("s")
        wid = sid * _NC + cid

        pltpu.sync_copy(dst_h.at[pl.ds(wid * _PER, _PER)], dst_v)
        pltpu.sync_copy(ones_h, ones_v)
        pltpu.sync_copy(zeros_h, acc.at[pl.ds(sid * _RPT, _RPT)])
        plsc.subcore_barrier()

        @pl.loop(0, _PER)
        def _edges(j):
            pltpu.sync_copy(ones_v, acc.at[dst_v.at[j]], add=True)
        plsc.subcore_barrier()

        pltpu.sync_copy(acc.at[pl.ds(sid * _RPT, _RPT)],
                        out.at[cid, pl.ds(sid * _RPT, _RPT)])
        plsc.subcore_barrier()

    return pl.kernel(
        body,
        out_type=jax.ShapeDtypeStruct((_NC, _NP, _W), jnp.float32),
        mesh=mesh,
        compiler_params=pltpu.CompilerParams(use_tc_tiling_on_sc=False),
        scratch_types=[
            pltpu.VMEM((_PER, _CHUNK), jnp.int32),
            pltpu.VMEM((_CHUNK, _W), jnp.float32),
            pltpu.VMEM_SHARED((_NP, _W), jnp.float32),
            pltpu.SemaphoreType.DMA,
        ],
    )


def _full(shape):
    return pl.BlockSpec(shape, lambda i: tuple(0 for _ in shape))


def _rows(*shape):
    nl = len(shape)
    return pl.BlockSpec(shape, lambda i: tuple(0 for _ in range(nl - 2)) + (i, 0))


def _tc0(degp, x):
    def body(degp_ref, x_ref, dinv_ref, g1_ref):
        ind = degp_ref[0, :, 0:1] + degp_ref[1, :, 0:1]
        dinv = lax.rsqrt(ind + 1.0)
        dinv_ref[...] = dinv
        g1_ref[...] = x_ref[...] * dinv

    return pl.pallas_call(
        body,
        grid=(_G,),
        in_specs=[_rows(_NC, _RB, _W), _rows(_RB, 8)],
        out_specs=[_rows(_RB, 1), _rows(_RB, 8)],
        out_shape=[jax.ShapeDtypeStruct((_NP, 1), jnp.float32),
                   jax.ShapeDtypeStruct((_NP, 8), jnp.float32)],
    )(degp, x)


def _dense_stage(parts, gs, dinv, mats, w_out, relu, pre_mm, n_next):
    C = len(gs)

    def body(*refs):
        p_refs = refs[:C]
        g_refs = refs[C:2 * C]
        dinv_ref = refs[2 * C]
        mat_refs = refs[2 * C + 1: 2 * C + 1 + len(mats)]
        out_refs = refs[2 * C + 1 + len(mats):]
        dinv = dinv_ref[...]
        s = jnp.concatenate(
            [p_refs[ci][0] + p_refs[ci][1] + g_refs[ci][...]
             for ci in range(C)], axis=1)
        u = dinv * s
        W, b = mat_refs[0][...], mat_refs[1][...]
        h = jnp.dot(u, W, preferred_element_type=jnp.float32) + b
        if relu:
            h = jnp.maximum(h, 0.0)
        t = h
        if pre_mm:
            t = jnp.dot(h, mat_refs[2][...], preferred_element_type=jnp.float32)
        out_refs[0][...] = h
        for cj in range(n_next):
            out_refs[1 + cj][...] = dinv * t[:, cj * _W:(cj + 1) * _W]

    in_specs = ([_rows(_NC, _RB, _W)] * C + [_rows(_RB, _W)] * C
                + [_rows(_RB, 1)] + [_full(m.shape) for m in mats])
    out_specs = [_rows(_RB, w_out)] + [_rows(_RB, _W)] * n_next
    out_shape = ([jax.ShapeDtypeStruct((_NP, w_out), jnp.float32)]
                 + [jax.ShapeDtypeStruct((_NP, _W), jnp.float32)] * n_next)
    return pl.pallas_call(
        body, grid=(_G,), in_specs=in_specs, out_specs=out_specs,
        out_shape=out_shape,
    )(*parts, *gs, dinv, *mats)


def _tc_pool(parts, gs, dinv, b4, batch2d):
    C = len(gs)
    neg = float("-inf")

    def body(*refs):
        p_refs = refs[:C]
        g_refs = refs[C:2 * C]
        dinv_ref, b4_ref, batch_ref, pool_ref = refs[2 * C:]
        i = pl.program_id(0)

        @pl.when(i == 0)
        def _init():
            pool_ref[...] = jnp.full((_NG, 64), neg, jnp.float32)

        dinv = dinv_ref[...]
        s = jnp.concatenate(
            [p_refs[ci][0] + p_refs[ci][1] + g_refs[ci][...]
             for ci in range(C)], axis=1)
        h4 = dinv * s + b4_ref[...]
        rowid = i * _RB + lax.broadcasted_iota(jnp.int32, (_RB, 1), 0)
        h4 = jnp.where(rowid < _N, h4, neg)
        gidx = batch_ref[...]
        lo = batch_ref[0, 0]
        hi = batch_ref[_RB - 1, 0]
        rows = lax.broadcasted_iota(jnp.int32, (_NG, 64), 0)

        def gbody(g, _):
            m = gidx == g
            v = jnp.max(jnp.where(m, h4, neg), axis=0, keepdims=True)
            cur = pool_ref[...]
            pool_ref[...] = jnp.where(rows == g, jnp.maximum(cur, v), cur)
            return 0

        lax.fori_loop(lo, hi + 1, gbody, 0)

    return pl.pallas_call(
        body,
        grid=(_G,),
        in_specs=([_rows(_NC, _RB, _W)] * C + [_rows(_RB, _W)] * C
                  + [_rows(_RB, 1), _full((1, 64)), _rows(_RB, 1)]),
        out_specs=pl.BlockSpec((_NG, 64), lambda i: (0, 0)),
        out_shape=jax.ShapeDtypeStruct((_NG, 64), jnp.float32),
    )(*parts, *gs, dinv, b4, batch2d)


def _tc_head(pooled, Wl, bl):
    def body(pool_ref, wl_ref, bl_ref, out_ref):
        z = jnp.dot(pool_ref[...], wl_ref[...],
                    preferred_element_type=jnp.float32) + bl_ref[...]
        m = jnp.max(z, axis=1, keepdims=True)
        lse = jnp.log(jnp.sum(jnp.exp(z - m), axis=1, keepdims=True)) + m
        out_ref[...] = z - lse

    return pl.pallas_call(
        body,
        grid=(1,),
        in_specs=[_full((_NG, 64)), _full((64, 2)), _full((1, 2))],
        out_specs=_full((_NG, 2)),
        out_shape=jax.ShapeDtypeStruct((_NG, 2), jnp.float32),
    )(pooled, Wl, bl.reshape(1, 2))


@jax.jit
def kernel(x, edge_index, batch_index, W1, b1, W2, b2, W3, b3, W4, b4, Wl, bl):
    pad_e = _NCH * _CHUNK - _E
    src = jnp.concatenate(
        [edge_index[0], jnp.zeros((pad_e,), jnp.int32)]).reshape(_NCH, _CHUNK)
    dst = jnp.concatenate(
        [edge_index[1], jnp.full((pad_e,), _N, jnp.int32)]).reshape(_NCH, _CHUNK)
    xp = jnp.concatenate([x, jnp.zeros((_NP - _N, 8), jnp.float32)], axis=0)
    batch2d = jnp.concatenate(
        [batch_index, jnp.full((_NP - _N,), _NG - 1, jnp.int32)]).reshape(_NP, 1)
    ones_h = jnp.ones((_CHUNK, _W), jnp.float32)
    zeros_h = jnp.zeros((_RPT, _W), jnp.float32)

    degp = _make_deg()(dst, ones_h, zeros_h)
    dinv, g1 = _tc0(degp, xp)

    p1 = _make_agg(1)(g1, src, dst, zeros_h)
    r1 = _dense_stage(
        p1, [g1], dinv, [W1, b1.reshape(1, 64)],
        w_out=64, relu=True, pre_mm=False, n_next=8)
    h1, g2 = r1[0], r1[1:]

    p2 = _make_agg(8)(*g2, src, dst, zeros_h)
    r2 = _dense_stage(
        p2, g2, dinv, [W2, b2.reshape(1, 128)],
        w_out=128, relu=True, pre_mm=False, n_next=16)
    h2, g3 = r2[0], r2[1:]

    p3 = _make_agg(16)(*g3, src, dst, zeros_h)
    r3 = _dense_stage(
        p3, g3, dinv, [W3, b3.reshape(1, 128), W4],
        w_out=128, relu=True, pre_mm=True, n_next=8)
    h3, g4 = r3[0], r3[1:]

    p4 = _make_agg(8)(*g4, src, dst, zeros_h)
    pooled = _tc_pool(p4, list(g4), dinv, b4.reshape(1, 64), batch2d)

    return _tc_head(pooled, Wl, bl)


# speedup vs baseline: 1.1459x; 1.1459x over previous
import jax
import jax.numpy as jnp
from jax import lax
from jax.experimental import pallas as pl
from jax.experimental.pallas import tpu as pltpu
from jax.experimental.pallas import tpu_sc as plsc

_NC, _NS = 2, 16
_CHUNK = 256
_W = 8
_N = 50000
_E = 1600000
_NG = 128
_NP = 50048
_NCH = -(-(_E // _CHUNK) // 32) * 32
_PER = _NCH // (_NC * _NS)
_RPT = _NP // _NS
_K = 8
_RB = 544
_G = _NP // _RB


def _make_agg(n_chunks_feat):
    mesh = plsc.VectorSubcoreMesh(core_axis_name="c", subcore_axis_name="s")
    nf = n_chunks_feat

    def body(*refs):
        tabs = refs[:nf]
        src_h, dst_h, zeros_h = refs[nf:nf + 3]
        outs = refs[nf + 3:2 * nf + 3]
        src_v, dst_v, rows0, rows1, acc, sem0, sem1 = refs[2 * nf + 3:]

        cid = lax.axis_index("c")
        sid = lax.axis_index("s")
        wid = sid * _NC + cid

        pltpu.sync_copy(src_h.at[pl.ds(wid * _PER, _PER)], src_v)
        pltpu.sync_copy(dst_h.at[pl.ds(wid * _PER, _PER)], dst_v)

        for ci in range(nf):
            pltpu.sync_copy(zeros_h, acc.at[pl.ds(sid * _RPT, _RPT)])
            plsc.subcore_barrier()

            pltpu.async_copy(tabs[ci].at[src_v.at[0]], rows0, sem0)

            @pl.loop(0, _PER, step=2)
            def _edges(j):
                pltpu.make_async_copy(
                    tabs[ci].at[src_v.at[j]], rows0, sem0).wait()
                pltpu.async_copy(tabs[ci].at[src_v.at[j + 1]], rows1, sem1)
                pltpu.sync_copy(rows0, acc.at[dst_v.at[j]], add=True)
                pltpu.make_async_copy(
                    tabs[ci].at[src_v.at[j + 1]], rows1, sem1).wait()

                @pl.when(j + 2 < _PER)
                def _next():
                    pltpu.async_copy(
                        tabs[ci].at[src_v.at[j + 2]], rows0, sem0)
                pltpu.sync_copy(rows1, acc.at[dst_v.at[j + 1]], add=True)
            plsc.subcore_barrier()

            pltpu.sync_copy(
                acc.at[pl.ds(sid * _RPT, _RPT)],
                outs[ci].at[cid, pl.ds(sid * _RPT, _RPT)])
            plsc.subcore_barrier()

    return pl.kernel(
        body,
        out_type=[jax.ShapeDtypeStruct((_NC, _NP, _W), jnp.float32)
                  for _ in range(nf)],
        mesh=mesh,
        compiler_params=pltpu.CompilerParams(use_tc_tiling_on_sc=False),
        scratch_types=[
            pltpu.VMEM((_PER, _CHUNK), jnp.int32),
            pltpu.VMEM((_PER, _CHUNK), jnp.int32),
            pltpu.VMEM((_CHUNK, _W), jnp.float32),
            pltpu.VMEM((_CHUNK, _W), jnp.float32),
            pltpu.VMEM_SHARED((_NP, _W), jnp.float32),
            pltpu.SemaphoreType.DMA,
            pltpu.SemaphoreType.DMA,
        ],
    )


def _make_deg():
    mesh = plsc.VectorSubcoreMesh(core_axis_name="c", subcore_axis_name="s")

    def body(dst_h, ones_h, zeros_h, out, dst_v, ones_v, acc, sem):
        del sem
        cid = lax.axis_index("c")
        sid = lax.axis_index("s")
        wid = sid * _NC + cid

        pltpu.sync_copy(dst_h.at[pl.ds(wid * _PER, _PER)], dst_v)
        pltpu.sync_copy(ones_h, ones_v)
        pltpu.sync_copy(zeros_h, acc.at[pl.ds(sid * _RPT, _RPT)])
        plsc.subcore_barrier()

        @pl.loop(0, _PER)
        def _edges(j):
            pltpu.sync_copy(ones_v, acc.at[dst_v.at[j]], add=True)
        plsc.subcore_barrier()

        pltpu.sync_copy(acc.at[pl.ds(sid * _RPT, _RPT)],
                        out.at[cid, pl.ds(sid * _RPT, _RPT)])
        plsc.subcore_barrier()

    return pl.kernel(
        body,
        out_type=jax.ShapeDtypeStruct((_NC, _NP, _W), jnp.float32),
        mesh=mesh,
        compiler_params=pltpu.CompilerParams(use_tc_tiling_on_sc=False),
        scratch_types=[
            pltpu.VMEM((_PER, _CHUNK), jnp.int32),
            pltpu.VMEM((_CHUNK, _W), jnp.float32),
            pltpu.VMEM_SHARED((_NP, _W), jnp.float32),
            pltpu.SemaphoreType.DMA,
        ],
    )


def _full(shape):
    return pl.BlockSpec(shape, lambda i: tuple(0 for _ in shape))


def _rows(*shape):
    nl = len(shape)
    return pl.BlockSpec(shape, lambda i: tuple(0 for _ in range(nl - 2)) + (i, 0))


def _tc0(degp, x):
    def body(degp_ref, x_ref, dinv_ref, g1_ref):
        ind = degp_ref[0, :, 0:1] + degp_ref[1, :, 0:1]
        dinv = lax.rsqrt(ind + 1.0)
        dinv_ref[...] = dinv
        g1_ref[...] = x_ref[...] * dinv

    return pl.pallas_call(
        body,
        grid=(_G,),
        in_specs=[_rows(_NC, _RB, _W), _rows(_RB, 8)],
        out_specs=[_rows(_RB, 1), _rows(_RB, 8)],
        out_shape=[jax.ShapeDtypeStruct((_NP, 1), jnp.float32),
                   jax.ShapeDtypeStruct((_NP, 8), jnp.float32)],
    )(degp, x)


def _dense_stage(parts, gs, dinv, mats, w_out, relu, pre_mm, n_next):
    C = len(gs)

    def body(*refs):
        p_refs = refs[:C]
        g_refs = refs[C:2 * C]
        dinv_ref = refs[2 * C]
        mat_refs = refs[2 * C + 1: 2 * C + 1 + len(mats)]
        out_refs = refs[2 * C + 1 + len(mats):]
        dinv = dinv_ref[...]
        s = jnp.concatenate(
            [p_refs[ci][0] + p_refs[ci][1] + g_refs[ci][...]
             for ci in range(C)], axis=1)
        u = dinv * s
        W, b = mat_refs[0][...], mat_refs[1][...]
        h = jnp.dot(u, W, preferred_element_type=jnp.float32) + b
        if relu:
            h = jnp.maximum(h, 0.0)
        t = h
        if pre_mm:
            t = jnp.dot(h, mat_refs[2][...], preferred_element_type=jnp.float32)
        out_refs[0][...] = h
        for cj in range(n_next):
            out_refs[1 + cj][...] = dinv * t[:, cj * _W:(cj + 1) * _W]

    in_specs = ([_rows(_NC, _RB, _W)] * C + [_rows(_RB, _W)] * C
                + [_rows(_RB, 1)] + [_full(m.shape) for m in mats])
    out_specs = [_rows(_RB, w_out)] + [_rows(_RB, _W)] * n_next
    out_shape = ([jax.ShapeDtypeStruct((_NP, w_out), jnp.float32)]
                 + [jax.ShapeDtypeStruct((_NP, _W), jnp.float32)] * n_next)
    return pl.pallas_call(
        body, grid=(_G,), in_specs=in_specs, out_specs=out_specs,
        out_shape=out_shape,
    )(*parts, *gs, dinv, *mats)


def _tc_pool(parts, gs, dinv, b4, batch2d):
    C = len(gs)
    neg = float("-inf")

    def body(*refs):
        p_refs = refs[:C]
        g_refs = refs[C:2 * C]
        dinv_ref, b4_ref, batch_ref, pool_ref = refs[2 * C:]
        i = pl.program_id(0)

        @pl.when(i == 0)
        def _init():
            pool_ref[...] = jnp.full((_NG, 64), neg, jnp.float32)

        dinv = dinv_ref[...]
        s = jnp.concatenate(
            [p_refs[ci][0] + p_refs[ci][1] + g_refs[ci][...]
             for ci in range(C)], axis=1)
        h4 = dinv * s + b4_ref[...]
        rowid = i * _RB + lax.broadcasted_iota(jnp.int32, (_RB, 1), 0)
        h4 = jnp.where(rowid < _N, h4, neg)
        gidx = batch_ref[...]
        lo = batch_ref[0, 0]
        hi = batch_ref[_RB - 1, 0]
        rows = lax.broadcasted_iota(jnp.int32, (_NG, 64), 0)

        def gbody(g, _):
            m = gidx == g
            v = jnp.max(jnp.where(m, h4, neg), axis=0, keepdims=True)
            cur = pool_ref[...]
            pool_ref[...] = jnp.where(rows == g, jnp.maximum(cur, v), cur)
            return 0

        lax.fori_loop(lo, hi + 1, gbody, 0)

    return pl.pallas_call(
        body,
        grid=(_G,),
        in_specs=([_rows(_NC, _RB, _W)] * C + [_rows(_RB, _W)] * C
                  + [_rows(_RB, 1), _full((1, 64)), _rows(_RB, 1)]),
        out_specs=pl.BlockSpec((_NG, 64), lambda i: (0, 0)),
        out_shape=jax.ShapeDtypeStruct((_NG, 64), jnp.float32),
    )(*parts, *gs, dinv, b4, batch2d)


def _tc_head(pooled, Wl, bl):
    def body(pool_ref, wl_ref, bl_ref, out_ref):
        z = jnp.dot(pool_ref[...], wl_ref[...],
                    preferred_element_type=jnp.float32) + bl_ref[...]
        m = jnp.max(z, axis=1, keepdims=True)
        lse = jnp.log(jnp.sum(jnp.exp(z - m), axis=1, keepdims=True)) + m
        out_ref[...] = z - lse

    return pl.pallas_call(
        body,
        grid=(1,),
        in_specs=[_full((_NG, 64)), _full((64, 2)), _full((1, 2))],
        out_specs=_full((_NG, 2)),
        out_shape=jax.ShapeDtypeStruct((_NG, 2), jnp.float32),
    )(pooled, Wl, bl.reshape(1, 2))


@jax.jit
def kernel(x, edge_index, batch_index, W1, b1, W2, b2, W3, b3, W4, b4, Wl, bl):
    pad_e = _NCH * _CHUNK - _E
    src = jnp.concatenate(
        [edge_index[0], jnp.zeros((pad_e,), jnp.int32)]).reshape(_NCH, _CHUNK)
    dst = jnp.concatenate(
        [edge_index[1], jnp.full((pad_e,), _N, jnp.int32)]).reshape(_NCH, _CHUNK)
    xp = jnp.concatenate([x, jnp.zeros((_NP - _N, 8), jnp.float32)], axis=0)
    batch2d = jnp.concatenate(
        [batch_index, jnp.full((_NP - _N,), _NG - 1, jnp.int32)]).reshape(_NP, 1)
    ones_h = jnp.ones((_CHUNK, _W), jnp.float32)
    zeros_h = jnp.zeros((_RPT, _W), jnp.float32)

    degp = _make_deg()(dst, ones_h, zeros_h)
    dinv, g1 = _tc0(degp, xp)

    p1 = _make_agg(1)(g1, src, dst, zeros_h)
    r1 = _dense_stage(
        p1, [g1], dinv, [W1, b1.reshape(1, 64)],
        w_out=64, relu=True, pre_mm=False, n_next=8)
    h1, g2 = r1[0], r1[1:]

    p2 = _make_agg(8)(*g2, src, dst, zeros_h)
    r2 = _dense_stage(
        p2, g2, dinv, [W2, b2.reshape(1, 128)],
        w_out=128, relu=True, pre_mm=False, n_next=16)
    h2, g3 = r2[0], r2[1:]

    p3 = _make_agg(16)(*g3, src, dst, zeros_h)
    r3 = _dense_stage(
        p3, g3, dinv, [W3, b3.reshape(1, 128), W4],
        w_out=128, relu=True, pre_mm=True, n_next=8)
    h3, g4 = r3[0], r3[1:]

    p4 = _make_agg(8)(*g4, src, dst, zeros_h)
    pooled = _tc_pool(p4, list(g4), dinv, b4.reshape(1, 64), batch2d)

    return _tc_head(pooled, Wl, bl)
